# baseline (speedup 1.0000x reference)
import functools

import jax
import jax.numpy as jnp
from jax import lax
from jax.experimental import pallas as pl
from jax.experimental.pallas import tpu as pltpu

_LN_EPS = 1e-5


def _fused_kernel(x_ref, wkv_ref, bkvt_ref, wq_ref, bq_ref,
                  wo_ref, bo_ref, gkt_ref, bkt_ref, gv_ref, bvl_ref,
                  out_ref, ka_ref, va_ref, *, nhead, d, inv_n, ntiles):
    n = x_ref.shape[1]
    hd = nhead * d
    nchunk = 2 * nhead
    stride = d + 16
    inv_d = 1.0 / float(d)
    n_f = float(n)
    tn = n // ntiles

    x_bf = x_ref[0].astype(jnp.bfloat16)

    for t in range(ntiles):
        xs = x_bf[t * tn:(t + 1) * tn, :]
        kv = lax.dot_general(wkv_ref[...], xs, (((0,), (1,)), ((), ())),
                             preferred_element_type=jnp.float32)
        kvb = kv + bkvt_ref[...]
        kv_bf = kvb[:2 * hd, :].astype(jnp.bfloat16)
        m = kvb[2 * hd:, :]

        sumsq = jnp.concatenate(
            [jnp.sum(jnp.square(kvb[c * d:(c + 1) * d, :]),
                     axis=0, keepdims=True)
             for c in range(nchunk)], axis=0)
        var = sumsq * inv_d - m * m
        r = lax.rsqrt(var + _LN_EPS)
        r_bf = r.astype(jnp.bfloat16)
        p_bf = (r * m).astype(jnp.bfloat16)

        ones_row = jnp.ones((1, tn), jnp.bfloat16)
        zero_pad = jnp.zeros((stride - d - 2, tn), jnp.bfloat16)
        for c in range(nchunk):
            dst = ka_ref if c < nhead else va_ref
            base = (c % nhead) * stride
            sl = pl.ds(t * tn, tn)
            dst[base:base + d, sl] = (kv_bf[c * d:(c + 1) * d, :]
                                      * r_bf[c:c + 1, :])
            dst[base + d:base + stride, sl] = jnp.concatenate(
                [p_bf[c:c + 1, :], ones_row, zero_pad], axis=0)

    m_parts = []
    for h in range(nhead):
        ka = ka_ref[h * stride:(h + 1) * stride, :]
        va = va_ref[h * stride:(h + 1) * stride, :]
        s_aug = lax.dot_general(ka, va, (((1,), (1,)), ((), ())),
                                preferred_element_type=jnp.float32)
        s = s_aug[:d, :d]
        c1 = s_aug[:d, d:d + 1]
        k1 = s_aug[:d, d + 1:d + 2]
        c2 = s_aug[d:d + 1, :d]
        one_v = s_aug[d + 1:d + 2, :d]
        pp = s_aug[d:d + 1, d:d + 1]
        s_pk = s_aug[d:d + 1, d + 1:d + 2]
        s_pv = s_aug[d + 1:d + 2, d:d + 1]

        gk_col = gkt_ref[:, h:h + 1]
        bk_col = bkt_ref[:, h:h + 1]
        gv_row = gv_ref[h:h + 1, :]
        bv_row = bvl_ref[h:h + 1, :]

        amap = ((s - c1 - c2 + pp) * (gk_col * gv_row)
                + (gk_col * (k1 - s_pk)) * bv_row
                + bk_col * ((one_v - s_pv) * gv_row)
                + n_f * (bk_col * bv_row))
        m_parts.append(jnp.dot(amap, wo_ref[h * d:(h + 1) * d, :],
                               preferred_element_type=jnp.float32))
    m_all = jnp.concatenate(m_parts, axis=0)

    w_eff = jnp.dot(wq_ref[...], m_all,
                    preferred_element_type=jnp.float32)
    b_eff = (jnp.dot(bq_ref[...], m_all,
                     preferred_element_type=jnp.float32)
             + bo_ref[...]) * inv_n

    out = jnp.dot(x_bf, (w_eff * inv_n).astype(jnp.bfloat16),
                  preferred_element_type=jnp.float32) + b_eff
    out_ref[0] = out.astype(out_ref.dtype)


def _full_spec(arr):
    zeros = (0,) * arr.ndim
    return pl.BlockSpec(arr.shape, lambda b: zeros)


def kernel(v, wq_t, bq, wk_t, bk, wv_t, bv, wo_t, bo, gk, bk_ln, gv, bv_ln):
    B, n, cin = v.shape
    nhead, d = gk.shape
    hd = nhead * d
    nchunk = 2 * nhead
    stride = d + 16
    inv_n = 1.0 / float(n)
    ntiles = 4 if n % 512 == 0 else (2 if n % 256 == 0 else 1)

    wkv = jnp.concatenate([wk_t, wv_t], axis=1)
    wmean = jnp.mean(wkv.reshape(cin, nchunk, d), axis=-1)
    wkv_aug = jnp.concatenate([wkv, wmean], axis=1).astype(jnp.bfloat16)
    bkv = jnp.concatenate([bk, bv], axis=1)
    bmean = jnp.mean(bkv.reshape(1, nchunk, d), axis=-1)
    bkvt_aug = jnp.concatenate([bkv, bmean], axis=1).T
    gkt = gk.T
    bkt = bk_ln.T

    flops = int(2 * B * n * cin * 2 * hd
                + 2 * B * n * hd * d
                + 2 * B * n * hd
                + 2 * B * hd * d * d
                + 2 * B * cin * hd * d
                + 2 * B * n * cin * d
                + 8 * B * n * hd)
    bytes_accessed = int(4 * (B * n * cin + B * n * d
                              + cin * hd + hd * d + 2 * hd + hd + d
                              + 4 * nhead * d))

    return pl.pallas_call(
        functools.partial(_fused_kernel, nhead=nhead, d=d, inv_n=inv_n,
                          ntiles=ntiles),
        out_shape=jax.ShapeDtypeStruct((B, n, d), jnp.float32),
        grid=(B,),
        in_specs=[pl.BlockSpec((1, n, cin), lambda b: (b, 0, 0)),
                  _full_spec(wkv_aug), _full_spec(bkvt_aug),
                  _full_spec(wq_t), _full_spec(bq),
                  _full_spec(wo_t), _full_spec(bo),
                  _full_spec(gkt), _full_spec(bkt),
                  _full_spec(gv), _full_spec(bv_ln)],
        out_specs=pl.BlockSpec((1, n, d), lambda b: (b, 0, 0)),
        scratch_shapes=[pltpu.VMEM((nhead * stride, n), jnp.bfloat16),
                        pltpu.VMEM((nhead * stride, n), jnp.bfloat16)],
        compiler_params=pltpu.CompilerParams(
            dimension_semantics=("parallel",)),
        cost_estimate=pl.CostEstimate(
            flops=flops, transcendentals=int(2 * B * n * nhead),
            bytes_accessed=bytes_accessed),
    )(v, wkv_aug, bkvt_aug, wq_t, bq, wo_t, bo, gkt, bkt, gv, bv_ln)


# speedup vs baseline: 1.4516x; 1.4516x over previous
import functools

import jax
import jax.numpy as jnp
from jax import lax
from jax.experimental import pallas as pl
from jax.experimental.pallas import tpu as pltpu

_LN_EPS = 1e-5


def _fused_kernel(x_ref, wkv_ref, bkvt_ref, a16_ref, wq_ref, bq_ref,
                  wo_ref, bo_ref, gkgv_ref, gkbv_ref, bkgv_ref, nbkbv_ref,
                  out_ref, ka_ref, va_ref, *, nhead, d, inv_n, ntiles):
    n = x_ref.shape[1]
    hd = nhead * d
    nchunk = 2 * nhead
    stride = d + 16
    inv_d = 1.0 / float(d)
    n_f = float(n)
    tn = n // ntiles

    x_bf = x_ref[0].astype(jnp.bfloat16)

    for t in range(ntiles):
        xs = x_bf[t * tn:(t + 1) * tn, :]
        kv = lax.dot_general(wkv_ref[...], xs, (((0,), (1,)), ((), ())),
                             preferred_element_type=jnp.float32)
        kvb = kv + bkvt_ref[...]
        kv_bf = kvb[:2 * hd, :].astype(jnp.bfloat16)
        m = kvb[2 * hd:, :]

        sumsq = jnp.dot(a16_ref[...], kv_bf * kv_bf,
                        preferred_element_type=jnp.float32)
        var = sumsq * inv_d - m * m
        r = lax.rsqrt(var + _LN_EPS)
        r_bf = r.astype(jnp.bfloat16)
        p_bf = (r * m).astype(jnp.bfloat16)

        ones_row = jnp.ones((1, tn), jnp.bfloat16)
        zero_pad = jnp.zeros((stride - d - 2, tn), jnp.bfloat16)
        for c in range(nchunk):
            dst = ka_ref if c < nhead else va_ref
            base = (c % nhead) * stride
            sl = pl.ds(t * tn, tn)
            dst[base:base + d, sl] = (kv_bf[c * d:(c + 1) * d, :]
                                      * r_bf[c:c + 1, :])
            dst[base + d:base + stride, sl] = jnp.concatenate(
                [p_bf[c:c + 1, :], ones_row, zero_pad], axis=0)

    s_augs = [lax.dot_general(ka_ref[h * stride:(h + 1) * stride, :],
                              va_ref[h * stride:(h + 1) * stride, :],
                              (((1,), (1,)), ((), ())),
                              preferred_element_type=jnp.float32)
              for h in range(nhead)]

    m_parts = []
    for h in range(nhead):
        s_aug = s_augs[h]
        s = s_aug[:d, :d]
        c1 = s_aug[:d, d:d + 1]
        k1 = s_aug[:d, d + 1:d + 2]
        c2 = s_aug[d:d + 1, :d]
        one_v = s_aug[d + 1:d + 2, :d]
        pp = s_aug[d:d + 1, d:d + 1]
        s_pk = s_aug[d:d + 1, d + 1:d + 2]
        s_pv = s_aug[d + 1:d + 2, d:d + 1]

        blk = slice(h * d, (h + 1) * d)
        amap = ((s - c1 - c2 + pp) * gkgv_ref[blk, :]
                + (k1 - s_pk) * gkbv_ref[blk, :]
                + (one_v - s_pv) * bkgv_ref[blk, :]
                + nbkbv_ref[blk, :])
        m_parts.append(jnp.dot(amap, wo_ref[blk, :],
                               preferred_element_type=jnp.float32))
    m_all = jnp.concatenate(m_parts, axis=0)

    w_eff = jnp.dot(wq_ref[...], m_all,
                    preferred_element_type=jnp.float32)
    b_eff = (jnp.dot(bq_ref[...], m_all,
                     preferred_element_type=jnp.float32)
             + bo_ref[...]) * inv_n

    out = jnp.dot(x_bf, (w_eff * inv_n).astype(jnp.bfloat16),
                  preferred_element_type=jnp.float32) + b_eff
    out_ref[0] = out.astype(out_ref.dtype)


def _full_spec(arr):
    zeros = (0,) * arr.ndim
    return pl.BlockSpec(arr.shape, lambda b: zeros)


def kernel(v, wq_t, bq, wk_t, bk, wv_t, bv, wo_t, bo, gk, bk_ln, gv, bv_ln):
    B, n, cin = v.shape
    nhead, d = gk.shape
    hd = nhead * d
    nchunk = 2 * nhead
    stride = d + 16
    inv_n = 1.0 / float(n)
    ntiles = 2 if n % 256 == 0 else 1

    wkv = jnp.concatenate([wk_t, wv_t], axis=1)
    wmean = jnp.mean(wkv.reshape(cin, nchunk, d), axis=-1)
    wkv_aug = jnp.concatenate([wkv, wmean], axis=1).astype(jnp.bfloat16)
    bkv = jnp.concatenate([bk, bv], axis=1)
    bmean = jnp.mean(bkv.reshape(1, nchunk, d), axis=-1)
    bkvt_aug = jnp.concatenate([bkv, bmean], axis=1).T
    a16 = jnp.kron(jnp.eye(nchunk, dtype=jnp.bfloat16),
                   jnp.ones((1, d), jnp.bfloat16))
    gk_c = gk.reshape(hd, 1)
    bk_c = bk_ln.reshape(hd, 1)
    gv_r = jnp.repeat(gv, d, axis=0)
    bv_r = jnp.repeat(bv_ln, d, axis=0)
    gkgv = gk_c * gv_r
    gkbv = gk_c * bv_r
    bkgv = bk_c * gv_r
    nbkbv = float(n) * (bk_c * bv_r)

    flops = int(2 * B * n * cin * 2 * hd
                + 2 * B * n * hd * d
                + 2 * B * n * hd
                + 2 * B * hd * d * d
                + 2 * B * cin * hd * d
                + 2 * B * n * cin * d
                + 8 * B * n * hd)
    bytes_accessed = int(4 * (B * n * cin + B * n * d
                              + cin * hd + hd * d + 2 * hd + hd + d
                              + 4 * nhead * d))

    return pl.pallas_call(
        functools.partial(_fused_kernel, nhead=nhead, d=d, inv_n=inv_n,
                          ntiles=ntiles),
        out_shape=jax.ShapeDtypeStruct((B, n, d), jnp.float32),
        grid=(B,),
        in_specs=[pl.BlockSpec((1, n, cin), lambda b: (b, 0, 0)),
                  _full_spec(wkv_aug), _full_spec(bkvt_aug),
                  _full_spec(a16),
                  _full_spec(wq_t), _full_spec(bq),
                  _full_spec(wo_t), _full_spec(bo),
                  _full_spec(gkgv), _full_spec(gkbv),
                  _full_spec(bkgv), _full_spec(nbkbv)],
        out_specs=pl.BlockSpec((1, n, d), lambda b: (b, 0, 0)),
        scratch_shapes=[pltpu.VMEM((nhead * stride, n), jnp.bfloat16),
                        pltpu.VMEM((nhead * stride, n), jnp.bfloat16)],
        compiler_params=pltpu.CompilerParams(
            dimension_semantics=("parallel",)),
        cost_estimate=pl.CostEstimate(
            flops=flops, transcendentals=int(2 * B * n * nhead),
            bytes_accessed=bytes_accessed),
    )(v, wkv_aug, bkvt_aug, a16, wq_t, bq, wo_t, bo, gkgv, gkbv, bkgv, nbkbv)


# speedup vs baseline: 1.5757x; 1.0855x over previous
import functools

import jax
import jax.numpy as jnp
from jax import lax
from jax.experimental import pallas as pl
from jax.experimental.pallas import tpu as pltpu

_LN_EPS = 1e-5


def _fused_kernel(x_ref, wkv_ref, bkvt_ref, a16_ref, wq_ref, bq_ref,
                  wo_ref, bo_ref, gkgv_ref, gkbv_ref, bkgv_ref, nbkbv_ref,
                  out_ref, ka_ref, va_ref, *, nhead, d, inv_n, ntiles):
    n = x_ref.shape[1]
    hd = nhead * d
    nchunk = 2 * nhead
    stride = d + 16
    inv_d = 1.0 / float(d)
    n_f = float(n)
    tn = n // ntiles

    x_bf = x_ref[0].astype(jnp.bfloat16)

    for t in range(ntiles):
        xs = x_bf[t * tn:(t + 1) * tn, :]
        kv = lax.dot_general(wkv_ref[...], xs, (((0,), (1,)), ((), ())),
                             preferred_element_type=jnp.float32)
        kvb = kv + bkvt_ref[...]
        kv_bf = kvb[:2 * hd, :].astype(jnp.bfloat16)
        m = kvb[2 * hd:, :]

        sq8 = (kv_bf * kv_bf).astype(jnp.float8_e4m3fn)
        sumsq = jnp.dot(a16_ref[...], sq8,
                        preferred_element_type=jnp.float32)
        var = sumsq * inv_d - m * m
        r = lax.rsqrt(var + _LN_EPS)
        r_bf = r.astype(jnp.bfloat16)
        p_bf = (r * m).astype(jnp.bfloat16)

        ones_row = jnp.ones((1, tn), jnp.bfloat16)
        zero_pad = jnp.zeros((stride - d - 2, tn), jnp.bfloat16)
        for c in range(nchunk):
            dst = ka_ref if c < nhead else va_ref
            base = (c % nhead) * stride
            sl = pl.ds(t * tn, tn)
            dst[base:base + d, sl] = (kv_bf[c * d:(c + 1) * d, :]
                                      * r_bf[c:c + 1, :])
            dst[base + d:base + stride, sl] = jnp.concatenate(
                [p_bf[c:c + 1, :], ones_row, zero_pad], axis=0)

    s_augs = [lax.dot_general(ka_ref[h * stride:(h + 1) * stride, :],
                              va_ref[h * stride:(h + 1) * stride, :],
                              (((1,), (1,)), ((), ())),
                              preferred_element_type=jnp.float32)
              for h in range(nhead)]

    m_parts = []
    for h in range(nhead):
        s_aug = s_augs[h]
        s = s_aug[:d, :d]
        c1 = s_aug[:d, d:d + 1]
        k1 = s_aug[:d, d + 1:d + 2]
        c2 = s_aug[d:d + 1, :d]
        one_v = s_aug[d + 1:d + 2, :d]
        pp = s_aug[d:d + 1, d:d + 1]
        s_pk = s_aug[d:d + 1, d + 1:d + 2]
        s_pv = s_aug[d + 1:d + 2, d:d + 1]

        blk = slice(h * d, (h + 1) * d)
        amap = ((s - c1 - c2 + pp) * gkgv_ref[blk, :]
                + (k1 - s_pk) * gkbv_ref[blk, :]
                + (one_v - s_pv) * bkgv_ref[blk, :]
                + nbkbv_ref[blk, :])
        m_parts.append(jnp.dot(amap, wo_ref[blk, :],
                               preferred_element_type=jnp.float32))
    m_all = jnp.concatenate(m_parts, axis=0)

    w_eff = jnp.dot(wq_ref[...], m_all,
                    preferred_element_type=jnp.float32)
    b_eff = (jnp.dot(bq_ref[...], m_all,
                     preferred_element_type=jnp.float32)
             + bo_ref[...]) * inv_n

    out = jnp.dot(x_bf, (w_eff * inv_n).astype(jnp.bfloat16),
                  preferred_element_type=jnp.float32) + b_eff
    out_ref[0] = out.astype(out_ref.dtype)


def _full_spec(arr):
    zeros = (0,) * arr.ndim
    return pl.BlockSpec(arr.shape, lambda b: zeros)


def kernel(v, wq_t, bq, wk_t, bk, wv_t, bv, wo_t, bo, gk, bk_ln, gv, bv_ln):
    B, n, cin = v.shape
    nhead, d = gk.shape
    hd = nhead * d
    nchunk = 2 * nhead
    stride = d + 16
    inv_n = 1.0 / float(n)
    ntiles = 2 if n % 256 == 0 else 1

    wkv = jnp.concatenate([wk_t, wv_t], axis=1)
    wmean = jnp.mean(wkv.reshape(cin, nchunk, d), axis=-1)
    wkv_aug = jnp.concatenate([wkv, wmean], axis=1).astype(jnp.bfloat16)
    bkv = jnp.concatenate([bk, bv], axis=1)
    bmean = jnp.mean(bkv.reshape(1, nchunk, d), axis=-1)
    bkvt_aug = jnp.concatenate([bkv, bmean], axis=1).T
    a16 = jnp.kron(jnp.eye(nchunk, dtype=jnp.float8_e4m3fn),
                   jnp.ones((1, d), jnp.float8_e4m3fn))
    gk_c = gk.reshape(hd, 1)
    bk_c = bk_ln.reshape(hd, 1)
    gv_r = jnp.repeat(gv, d, axis=0)
    bv_r = jnp.repeat(bv_ln, d, axis=0)
    gkgv = gk_c * gv_r
    gkbv = gk_c * bv_r
    bkgv = bk_c * gv_r
    nbkbv = float(n) * (bk_c * bv_r)

    flops = int(2 * B * n * cin * 2 * hd
                + 2 * B * n * hd * d
                + 2 * B * n * hd
                + 2 * B * hd * d * d
                + 2 * B * cin * hd * d
                + 2 * B * n * cin * d
                + 8 * B * n * hd)
    bytes_accessed = int(4 * (B * n * cin + B * n * d
                              + cin * hd + hd * d + 2 * hd + hd + d
                              + 4 * nhead * d))

    return pl.pallas_call(
        functools.partial(_fused_kernel, nhead=nhead, d=d, inv_n=inv_n,
                          ntiles=ntiles),
        out_shape=jax.ShapeDtypeStruct((B, n, d), jnp.float32),
        grid=(B,),
        in_specs=[pl.BlockSpec((1, n, cin), lambda b: (b, 0, 0)),
                  _full_spec(wkv_aug), _full_spec(bkvt_aug),
                  _full_spec(a16),
                  pl.BlockSpec(wq_t.shape, lambda b: (0, 0)),
                  _full_spec(bq),
                  pl.BlockSpec(wo_t.shape, lambda b: (0, 0)),
                  _full_spec(bo),
                  _full_spec(gkgv), _full_spec(gkbv),
                  _full_spec(bkgv), _full_spec(nbkbv)],
        out_specs=pl.BlockSpec((1, n, d), lambda b: (b, 0, 0)),
        scratch_shapes=[pltpu.VMEM((nhead * stride, n), jnp.bfloat16),
                        pltpu.VMEM((nhead * stride, n), jnp.bfloat16)],
        compiler_params=pltpu.CompilerParams(
            dimension_semantics=("parallel",)),
        cost_estimate=pl.CostEstimate(
            flops=flops, transcendentals=int(2 * B * n * nhead),
            bytes_accessed=bytes_accessed),
    )(v, wkv_aug, bkvt_aug, a16, wq_t, bq, wo_t, bo, gkgv, gkbv, bkgv, nbkbv)


# speedup vs baseline: 1.6615x; 1.0544x over previous
import functools

import jax
import jax.numpy as jnp
from jax import lax
from jax.experimental import pallas as pl
from jax.experimental.pallas import tpu as pltpu

_LN_EPS = 1e-5


def _fused_kernel(x_ref, wkv_ref, bkvt_ref, a16_ref, wq_ref, bq_ref,
                  wo_ref, bo_ref, gkgv_ref, gkbv_ref, bkgv_ref, nbkbv_ref,
                  out_ref, ka_ref, va_ref, x_scr, w_scr, b_scr,
                  *, nhead, d, inv_n, ntiles, nbatch):
    n = x_ref.shape[1]
    hd = nhead * d
    nchunk = 2 * nhead
    stride = d + 16
    inv_d = 1.0 / float(d)
    n_f = float(n)
    tn = n // ntiles

    j = pl.program_id(0)

    out = jnp.dot(x_scr[...], w_scr[...],
                  preferred_element_type=jnp.float32) + b_scr[0:1, :]
    out_ref[0] = out.astype(out_ref.dtype)

    @pl.when(j < nbatch)
    def _phase_a():
        x_bf = x_ref[0].astype(jnp.bfloat16)
        x_scr[...] = x_bf
        for t in range(ntiles):
            xs = x_bf[t * tn:(t + 1) * tn, :]
            kv = lax.dot_general(wkv_ref[...], xs, (((0,), (1,)), ((), ())),
                                 preferred_element_type=jnp.float32)
            kvb = kv + bkvt_ref[...]
            kv_bf = kvb[:2 * hd, :].astype(jnp.bfloat16)
            m = kvb[2 * hd:, :]

            sq8 = (kv_bf * kv_bf).astype(jnp.float8_e4m3fn)
            sumsq = jnp.dot(a16_ref[...], sq8,
                            preferred_element_type=jnp.float32)
            var = sumsq * inv_d - m * m
            r = lax.rsqrt(var + _LN_EPS)
            r_bf = r.astype(jnp.bfloat16)
            p_bf = (r * m).astype(jnp.bfloat16)

            ones_row = jnp.ones((1, tn), jnp.bfloat16)
            zero_pad = jnp.zeros((stride - d - 2, tn), jnp.bfloat16)
            for c in range(nchunk):
                dst = ka_ref if c < nhead else va_ref
                base = (c % nhead) * stride
                sl = pl.ds(t * tn, tn)
                dst[base:base + d, sl] = (kv_bf[c * d:(c + 1) * d, :]
                                          * r_bf[c:c + 1, :])
                dst[base + d:base + stride, sl] = jnp.concatenate(
                    [p_bf[c:c + 1, :], ones_row, zero_pad], axis=0)

        s_augs = [lax.dot_general(ka_ref[h * stride:(h + 1) * stride, :],
                                  va_ref[h * stride:(h + 1) * stride, :],
                                  (((1,), (1,)), ((), ())),
                                  preferred_element_type=jnp.float32)
                  for h in range(nhead)]

        m_parts = []
        for h in range(nhead):
            s_aug = s_augs[h]
            s = s_aug[:d, :d]
            c1 = s_aug[:d, d:d + 1]
            k1 = s_aug[:d, d + 1:d + 2]
            c2 = s_aug[d:d + 1, :d]
            one_v = s_aug[d + 1:d + 2, :d]
            pp = s_aug[d:d + 1, d:d + 1]
            s_pk = s_aug[d:d + 1, d + 1:d + 2]
            s_pv = s_aug[d + 1:d + 2, d:d + 1]

            blk = slice(h * d, (h + 1) * d)
            amap = ((s - c1 - c2 + pp) * gkgv_ref[blk, :]
                    + (k1 - s_pk) * gkbv_ref[blk, :]
                    + (one_v - s_pv) * bkgv_ref[blk, :]
                    + nbkbv_ref[blk, :])
            m_parts.append(jnp.dot(amap, wo_ref[blk, :],
                                   preferred_element_type=jnp.float32))
        m_all = jnp.concatenate(m_parts, axis=0)

        w_eff = jnp.dot(wq_ref[...], m_all,
                        preferred_element_type=jnp.float32)
        b_eff = (jnp.dot(bq_ref[...], m_all,
                         preferred_element_type=jnp.float32)
                 + bo_ref[...]) * inv_n
        w_scr[...] = (w_eff * inv_n).astype(jnp.bfloat16)
        b_scr[0:1, :] = b_eff


def _full_spec(arr):
    zeros = (0,) * arr.ndim
    return pl.BlockSpec(arr.shape, lambda b: zeros)


def kernel(v, wq_t, bq, wk_t, bk, wv_t, bv, wo_t, bo, gk, bk_ln, gv, bv_ln):
    B, n, cin = v.shape
    nhead, d = gk.shape
    hd = nhead * d
    nchunk = 2 * nhead
    stride = d + 16
    inv_n = 1.0 / float(n)
    ntiles = 2 if n % 256 == 0 else 1

    wkv = jnp.concatenate([wk_t, wv_t], axis=1)
    wmean = jnp.mean(wkv.reshape(cin, nchunk, d), axis=-1)
    wkv_aug = jnp.concatenate([wkv, wmean], axis=1).astype(jnp.bfloat16)
    bkv = jnp.concatenate([bk, bv], axis=1)
    bmean = jnp.mean(bkv.reshape(1, nchunk, d), axis=-1)
    bkvt_aug = jnp.concatenate([bkv, bmean], axis=1).T
    a16 = jnp.kron(jnp.eye(nchunk, dtype=jnp.float8_e4m3fn),
                   jnp.ones((1, d), jnp.float8_e4m3fn))
    gk_c = gk.reshape(hd, 1)
    bk_c = bk_ln.reshape(hd, 1)
    gv_r = jnp.repeat(gv, d, axis=0)
    bv_r = jnp.repeat(bv_ln, d, axis=0)
    gkgv = gk_c * gv_r
    gkbv = gk_c * bv_r
    bkgv = bk_c * gv_r
    nbkbv = float(n) * (bk_c * bv_r)

    flops = int(2 * B * n * cin * 2 * hd
                + 2 * B * n * hd * d
                + 2 * B * n * hd
                + 2 * B * hd * d * d
                + 2 * B * cin * hd * d
                + 2 * B * n * cin * d
                + 8 * B * n * hd)
    bytes_accessed = int(4 * (B * n * cin + B * n * d
                              + cin * hd + hd * d + 2 * hd + hd + d
                              + 4 * nhead * d))

    return pl.pallas_call(
        functools.partial(_fused_kernel, nhead=nhead, d=d, inv_n=inv_n,
                          ntiles=ntiles, nbatch=B),
        out_shape=jax.ShapeDtypeStruct((B, n, d), jnp.float32),
        grid=(B + 1,),
        in_specs=[pl.BlockSpec((1, n, cin),
                               lambda b: (jnp.minimum(b, B - 1), 0, 0)),
                  _full_spec(wkv_aug), _full_spec(bkvt_aug),
                  _full_spec(a16),
                  pl.BlockSpec(wq_t.shape, lambda b: (0, 0)),
                  _full_spec(bq),
                  pl.BlockSpec(wo_t.shape, lambda b: (0, 0)),
                  _full_spec(bo),
                  _full_spec(gkgv), _full_spec(gkbv),
                  _full_spec(bkgv), _full_spec(nbkbv)],
        out_specs=pl.BlockSpec((1, n, d),
                               lambda b: (jnp.maximum(b - 1, 0), 0, 0)),
        scratch_shapes=[pltpu.VMEM((nhead * stride, n), jnp.bfloat16),
                        pltpu.VMEM((nhead * stride, n), jnp.bfloat16),
                        pltpu.VMEM((n, cin), jnp.bfloat16),
                        pltpu.VMEM((cin, d), jnp.bfloat16),
                        pltpu.VMEM((8, d), jnp.float32)],
        compiler_params=pltpu.CompilerParams(
            dimension_semantics=("arbitrary",)),
        cost_estimate=pl.CostEstimate(
            flops=flops, transcendentals=int(2 * B * n * nhead),
            bytes_accessed=bytes_accessed),
    )(v, wkv_aug, bkvt_aug, a16, wq_t, bq, wo_t, bo, gkgv, gkbv, bkgv, nbkbv)
